# baseline (speedup 1.0000x reference)
import jax
import jax.numpy as jnp
from jax.experimental import pallas as pl

_BM = 400


def _xw_kernel(x_ref, w_ref, o_ref):
    o_ref[...] = jnp.dot(x_ref[...], w_ref[...],
                         preferred_element_type=jnp.float32)


def _layer1_kernel(adj_ref, s1_ref, b1_ref, w2_ref, o_ref):
    h = jnp.dot(adj_ref[...], s1_ref[...],
                preferred_element_type=jnp.float32) + b1_ref[...]
    h = jnp.maximum(h, 0.0)
    o_ref[...] = jnp.dot(h, w2_ref[...], preferred_element_type=jnp.float32)


def _layer2_kernel(adj_ref, s2_ref, b2_ref, o_ref):
    o_ref[...] = jnp.dot(adj_ref[...], s2_ref[...],
                         preferred_element_type=jnp.float32) + b2_ref[...]


def kernel(x, adj, W1, b1, W2, b2):
    n, nfeat = x.shape
    nhid = W1.shape[1]
    ncls = W2.shape[1]
    b1r = b1.reshape(1, nhid)
    b2r = b2.reshape(1, ncls)

    s1 = pl.pallas_call(
        _xw_kernel,
        out_shape=jax.ShapeDtypeStruct((n, nhid), jnp.float32),
    )(x, W1)

    grid = (n // _BM,)
    adj_spec = pl.BlockSpec((_BM, n), lambda i: (i, 0))

    s2 = pl.pallas_call(
        _layer1_kernel,
        grid=grid,
        in_specs=[
            adj_spec,
            pl.BlockSpec((n, nhid), lambda i: (0, 0)),
            pl.BlockSpec((1, nhid), lambda i: (0, 0)),
            pl.BlockSpec((nhid, ncls), lambda i: (0, 0)),
        ],
        out_specs=pl.BlockSpec((_BM, ncls), lambda i: (i, 0)),
        out_shape=jax.ShapeDtypeStruct((n, ncls), jnp.float32),
    )(adj, s1, b1r, W2)

    out = pl.pallas_call(
        _layer2_kernel,
        grid=grid,
        in_specs=[
            adj_spec,
            pl.BlockSpec((n, ncls), lambda i: (0, 0)),
            pl.BlockSpec((1, ncls), lambda i: (0, 0)),
        ],
        out_specs=pl.BlockSpec((_BM, ncls), lambda i: (i, 0)),
        out_shape=jax.ShapeDtypeStruct((n, ncls), jnp.float32),
    )(adj, s2, b2r)

    return out
